# baseline (speedup 1.0000x reference)
import jax
import jax.numpy as jnp
from jax.experimental import pallas as pl

_NUM_SKILL = 100
_EMB_DIM = 16
_K = 4
_LANES = _NUM_SKILL * _EMB_DIM
_ROWS_BLK = 512


def _concept_kernel(conc_ref, embflat_ref, out_ref):
    rows = conc_ref.shape[0]
    n_iota = jax.lax.broadcasted_iota(jnp.int32, (rows, _LANES), 1) // _EMB_DIM
    conc = conc_ref[...]
    counts = jnp.zeros((rows, _LANES), jnp.float32)
    for k in range(_K):
        counts += (n_iota == conc[:, k : k + 1]).astype(jnp.float32)
    out_ref[...] = counts * embflat_ref[...]


def kernel(concepts, emb_table_skill):
    b, s, k = concepts.shape
    rows = b * s
    conc2d = concepts.reshape(rows, k).astype(jnp.int32)
    embflat = emb_table_skill[:_NUM_SKILL].reshape(1, _LANES)

    grid = (rows // _ROWS_BLK,)
    out = pl.pallas_call(
        _concept_kernel,
        grid=grid,
        in_specs=[
            pl.BlockSpec((_ROWS_BLK, k), lambda i: (i, 0)),
            pl.BlockSpec((1, _LANES), lambda i: (0, 0)),
        ],
        out_specs=pl.BlockSpec((_ROWS_BLK, _LANES), lambda i: (i, 0)),
        out_shape=jax.ShapeDtypeStruct((rows, _LANES), jnp.float32),
    )(conc2d, embflat)
    return out.reshape(b, s, _NUM_SKILL, _EMB_DIM)


# speedup vs baseline: 8.2376x; 8.2376x over previous
import jax
import jax.numpy as jnp
from jax.experimental import pallas as pl

_NUM_SKILL = 100
_EMB_DIM = 16
_B_BLK = 128


def _concept_kernel(conc_ref, emb_bc_ref, out_ref):
    c0 = conc_ref[0, 0, :]
    c1 = conc_ref[0, 1, :]
    c2 = conc_ref[0, 2, :]
    c3 = conc_ref[0, 3, :]

    def body(n, _):
        cnt = (
            (c0 == n).astype(jnp.float32)
            + (c1 == n).astype(jnp.float32)
            + (c2 == n).astype(jnp.float32)
            + (c3 == n).astype(jnp.float32)
        )
        out_ref[0, n] = jnp.broadcast_to(cnt[None, :], (_EMB_DIM, _B_BLK)) * emb_bc_ref[n]
        return 0

    jax.lax.fori_loop(0, _NUM_SKILL, body, 0)


def kernel(concepts, emb_table_skill):
    b, s, k = concepts.shape
    conc_t = jnp.transpose(concepts.astype(jnp.int32), (1, 2, 0))
    emb_bc = jnp.broadcast_to(
        emb_table_skill[:_NUM_SKILL, :, None], (_NUM_SKILL, _EMB_DIM, _B_BLK)
    )

    out_t = pl.pallas_call(
        _concept_kernel,
        grid=(s, b // _B_BLK),
        in_specs=[
            pl.BlockSpec((1, k, _B_BLK), lambda i, j: (i, 0, j)),
            pl.BlockSpec((_NUM_SKILL, _EMB_DIM, _B_BLK), lambda i, j: (0, 0, 0)),
        ],
        out_specs=pl.BlockSpec((1, _NUM_SKILL, _EMB_DIM, _B_BLK), lambda i, j: (i, 0, 0, j)),
        out_shape=jax.ShapeDtypeStruct((s, _NUM_SKILL, _EMB_DIM, b), jnp.float32),
    )(conc_t, emb_bc)
    return jnp.transpose(out_t, (3, 0, 1, 2))


# speedup vs baseline: 24.8495x; 3.0166x over previous
import jax
import jax.numpy as jnp
from jax.experimental import pallas as pl

_NUM_SKILL = 100
_EMB_DIM = 16
_B_BLK = 1024


def _concept_kernel(conc_ref, emb_bc_ref, out_ref):
    c0 = conc_ref[0, 0, :]
    c1 = conc_ref[0, 1, :]
    c2 = conc_ref[0, 2, :]
    c3 = conc_ref[0, 3, :]

    def body(n, _):
        cnt = (
            (c0 == n).astype(jnp.float32)
            + (c1 == n).astype(jnp.float32)
            + (c2 == n).astype(jnp.float32)
            + (c3 == n).astype(jnp.float32)
        )
        out_ref[0, n] = jnp.broadcast_to(cnt[None, :], (_EMB_DIM, _B_BLK)) * emb_bc_ref[n]
        return 0

    jax.lax.fori_loop(0, _NUM_SKILL, body, 0)


def kernel(concepts, emb_table_skill):
    b, s, k = concepts.shape
    conc_t = jnp.transpose(concepts.astype(jnp.int32), (1, 2, 0))
    emb_bc = jnp.broadcast_to(
        emb_table_skill[:_NUM_SKILL, :, None], (_NUM_SKILL, _EMB_DIM, _B_BLK)
    )

    out_t = pl.pallas_call(
        _concept_kernel,
        grid=(s, b // _B_BLK),
        in_specs=[
            pl.BlockSpec((1, k, _B_BLK), lambda i, j: (i, 0, j)),
            pl.BlockSpec((_NUM_SKILL, _EMB_DIM, _B_BLK), lambda i, j: (0, 0, j)),
        ],
        out_specs=pl.BlockSpec((1, _NUM_SKILL, _EMB_DIM, _B_BLK), lambda i, j: (i, 0, 0, j)),
        out_shape=jax.ShapeDtypeStruct((s, _NUM_SKILL, _EMB_DIM, b), jnp.float32),
    )(conc_t, emb_bc)
    return jnp.transpose(out_t, (3, 0, 1, 2))
